# baseline (speedup 1.0000x reference)
import jax
import jax.numpy as jnp
import numpy as np
from jax import lax
from jax.experimental import pallas as pl
from jax.experimental.pallas import tpu as pltpu
from jax.experimental.pallas import tpu_sc as plsc

N = 10000
E = 320000
D_IN = 128
D_HID = 128
D_OUT = 3
BN_EPS = 1e-5

NC, NS = 2, 16
NW = NC * NS
CHUNK = 128
K = 2 * (-(-E // (NW * CHUNK * 2)))
KH = K // 2
EPW = K * CHUNK
EPAD = NW * EPW
NP = 10240
DEG_ROWS = 2048
W2P = 16

_mesh = plsc.VectorSubcoreMesh(core_axis_name="c", subcore_axis_name="s",
                               num_cores=NC, num_subcores=NS)


def _make_scat(width, acc_rows):
    rpt = acc_rows // NS

    def body(tab_hbm, gidx_hbm, sidx_hbm, z_hbm, out_hbm,
             gidx_v, sidx_v, rows_a, rows_b, acc_sh, sem_a, sem_b):
        c = lax.axis_index("c")
        s = lax.axis_index("s")
        wid = c * NS + s
        pltpu.sync_copy(z_hbm, acc_sh.at[pl.ds(s * rpt, rpt)])
        plsc.subcore_barrier()

        for h in range(2):
            pltpu.sync_copy(gidx_hbm.at[wid, pl.ds(h * KH, KH)], gidx_v)
            pltpu.sync_copy(sidx_hbm.at[wid, pl.ds(h * KH, KH)], sidx_v)
            pltpu.async_copy(tab_hbm.at[gidx_v.at[0]], rows_a, sem_a)

            def pair(i, carry):
                j0 = 2 * i
                j1 = j0 + 1
                pltpu.make_async_copy(tab_hbm.at[gidx_v.at[j0]], rows_a,
                                      sem_a).wait()
                pltpu.async_copy(tab_hbm.at[gidx_v.at[j1]], rows_b, sem_b)
                pltpu.sync_copy(rows_a, acc_sh.at[sidx_v.at[j0]], add=True)
                pltpu.make_async_copy(tab_hbm.at[gidx_v.at[j1]], rows_b,
                                      sem_b).wait()

                @pl.when(j1 + 1 < KH)
                def _():
                    pltpu.async_copy(tab_hbm.at[gidx_v.at[j1 + 1]], rows_a,
                                     sem_a)

                pltpu.sync_copy(rows_b, acc_sh.at[sidx_v.at[j1]], add=True)
                return carry

            lax.fori_loop(0, KH // 2, pair, 0)
        plsc.subcore_barrier()
        pltpu.sync_copy(acc_sh.at[pl.ds(s * rpt, rpt)],
                        out_hbm.at[c, pl.ds(s * rpt, rpt)])

    return pl.kernel(
        body,
        out_type=jax.ShapeDtypeStruct((NC, acc_rows, width), jnp.float32),
        mesh=_mesh,
        scratch_types=[
            pltpu.VMEM((KH, CHUNK), jnp.int32),
            pltpu.VMEM((KH, CHUNK), jnp.int32),
            pltpu.VMEM((CHUNK, width), jnp.float32),
            pltpu.VMEM((CHUNK, width), jnp.float32),
            pltpu.VMEM_SHARED((acc_rows, width), jnp.float32),
            pltpu.SemaphoreType.DMA,
            pltpu.SemaphoreType.DMA,
        ],
    )


_BLK = 2048


def _dinv_of(d0_ref, d1_ref):
    deg = d0_ref[...] + d1_ref[...] + 1.0
    return lax.rsqrt(deg)


def _tca_body(x_ref, d0_ref, d1_ref, xh_ref):
    xh_ref[...] = x_ref[...] * _dinv_of(d0_ref, d1_ref)


def _tcb_body(p1_ref, xh_ref, d0_ref, d1_ref, w1_ref, b1_ref, gam_ref,
              bet_ref, ah_ref):
    dinv = _dinv_of(d0_ref, d1_ref)
    z1 = (p1_ref[0] + p1_ref[1] + xh_ref[...]) * dinv
    o1 = jnp.dot(z1, w1_ref[...], preferred_element_type=jnp.float32) + b1_ref[...]
    a = jnp.maximum(o1, 0.0) * gam_ref[...] + bet_ref[...]
    ah_ref[...] = a * dinv


def _tcc_body(p2_ref, ah_ref, d0_ref, d1_ref, w2_ref, b2_ref, o_ref):
    dinv = _dinv_of(d0_ref, d1_ref)
    z2 = (p2_ref[0] + p2_ref[1] + ah_ref[...]) * dinv
    o_ref[...] = jnp.dot(z2, w2_ref[...],
                         preferred_element_type=jnp.float32) + b2_ref[...]


def _row_spec(w):
    return pl.BlockSpec((_BLK, w), lambda i: (i, 0))


def _pair_spec(w):
    return pl.BlockSpec((2, _BLK, w), lambda i: (0, i, 0))


def _full_spec(r, ccols):
    return pl.BlockSpec((r, ccols), lambda i: (0, 0))


def kernel(x, edge_index, W1, b1, gamma, beta, W2, b2):
    src = edge_index[0]
    dst = edge_index[1]
    pad = EPAD - E
    src_p = jnp.concatenate([src, jnp.zeros((pad,), jnp.int32)]).reshape(NW, K, CHUNK)
    dst_p = jnp.concatenate([dst, jnp.full((pad,), N, jnp.int32)]).reshape(NW, K, CHUNK)
    ridx = dst_p & 127
    qidx = dst_p >> 7

    ident = jnp.eye(CHUNK, dtype=jnp.float32)
    z_deg = jnp.zeros((DEG_ROWS // NS, CHUNK), jnp.float32)
    z128 = jnp.zeros((NP // NS, D_HID), jnp.float32)

    x_pad = jnp.pad(x, ((0, NP - N), (0, 0)))
    gamma_eff = (gamma * np.float32(1.0 / np.sqrt(1.0 + BN_EPS))).reshape(1, D_HID)
    beta_r = beta.reshape(1, D_HID)
    b1_r = b1.reshape(1, D_HID)
    W2p = jnp.zeros((D_HID, W2P), jnp.float32).at[:, :D_OUT].set(W2)
    b2p = jnp.zeros((1, W2P), jnp.float32).at[0, :D_OUT].set(b2)

    dp = _make_scat(CHUNK, DEG_ROWS)(ident, ridx, qidx, z_deg)
    dflat = dp.reshape(NC, DEG_ROWS * CHUNK)
    dd0 = dflat[0, :NP].reshape(NP, 1)
    dd1 = dflat[1, :NP].reshape(NP, 1)

    grid = NP // _BLK
    col_spec = pl.BlockSpec((_BLK, 1), lambda i: (i, 0))

    xh = pl.pallas_call(
        _tca_body,
        grid=(grid,),
        in_specs=[_row_spec(D_IN), col_spec, col_spec],
        out_specs=_row_spec(D_IN),
        out_shape=jax.ShapeDtypeStruct((NP, D_IN), jnp.float32),
    )(x_pad, dd0, dd1)

    p1 = _make_scat(D_HID, NP)(xh, src_p, dst_p, z128)

    ah = pl.pallas_call(
        _tcb_body,
        grid=(grid,),
        in_specs=[_pair_spec(D_HID), _row_spec(D_HID), col_spec, col_spec,
                  _full_spec(D_IN, D_HID), _full_spec(1, D_HID),
                  _full_spec(1, D_HID), _full_spec(1, D_HID)],
        out_specs=_row_spec(D_HID),
        out_shape=jax.ShapeDtypeStruct((NP, D_HID), jnp.float32),
    )(p1, xh, dd0, dd1, W1, b1_r, gamma_eff, beta_r)

    p2 = _make_scat(D_HID, NP)(ah, src_p, dst_p, z128)

    o = pl.pallas_call(
        _tcc_body,
        grid=(grid,),
        in_specs=[_pair_spec(D_HID), _row_spec(D_HID), col_spec, col_spec,
                  _full_spec(D_HID, W2P), _full_spec(1, W2P)],
        out_specs=_row_spec(W2P),
        out_shape=jax.ShapeDtypeStruct((NP, W2P), jnp.float32),
    )(p2, ah, dd0, dd1, W2p, b2p)

    return o[:N, :D_OUT]


# speedup vs baseline: 1.7625x; 1.7625x over previous
import jax
import jax.numpy as jnp
import numpy as np
from jax import lax
from jax.experimental import pallas as pl
from jax.experimental.pallas import tpu as pltpu
from jax.experimental.pallas import tpu_sc as plsc

N = 10000
E = 320000
D_IN = 128
D_HID = 128
D_OUT = 3
BN_EPS = 1e-5

NC, NS = 2, 16
NW = NC * NS
CHUNK = 128
K = -(-E // (NW * CHUNK))
EPW = K * CHUNK
EPAD = NW * EPW
NP = 10240
W2P = 16
HI = NP // CHUNK

_mesh = plsc.VectorSubcoreMesh(core_axis_name="c", subcore_axis_name="s",
                               num_cores=NC, num_subcores=NS)


def _make_scat(width, acc_rows):
    rpt = acc_rows // NS

    def body(tab_hbm, gidx_hbm, sidx_hbm, z_hbm, out_hbm,
             gidx_v, sidx_v, rows_v, acc_sh, sem):
        c = lax.axis_index("c")
        s = lax.axis_index("s")
        wid = c * NS + s
        pltpu.sync_copy(z_hbm, acc_sh.at[pl.ds(s * rpt, rpt)])
        pltpu.sync_copy(gidx_hbm.at[wid], gidx_v)
        pltpu.sync_copy(sidx_hbm.at[wid], sidx_v)
        plsc.subcore_barrier()

        def chunk(j, carry):
            pltpu.async_copy(tab_hbm.at[gidx_v.at[j]], rows_v, sem).wait()
            pltpu.sync_copy(rows_v, acc_sh.at[sidx_v.at[j]], add=True)
            return carry

        lax.fori_loop(0, K, chunk, 0)
        plsc.subcore_barrier()
        pltpu.sync_copy(acc_sh.at[pl.ds(s * rpt, rpt)],
                        out_hbm.at[c, pl.ds(s * rpt, rpt)])

    return pl.kernel(
        body,
        out_type=jax.ShapeDtypeStruct((NC, acc_rows, width), jnp.float32),
        mesh=_mesh,
        scratch_types=[
            pltpu.VMEM((K, CHUNK), jnp.int32),
            pltpu.VMEM((K, CHUNK), jnp.int32),
            pltpu.VMEM((CHUNK, width), jnp.float32),
            pltpu.VMEM_SHARED((acc_rows, width), jnp.float32),
            pltpu.SemaphoreType.DMA,
        ],
    )


_EB = 3200


def _deg_body(hi_ref, lo_ref, deg_ref):
    i = pl.program_id(0)
    hia = jnp.broadcast_to(hi_ref[...], (HI, _EB)) == lax.broadcasted_iota(
        jnp.int32, (HI, _EB), 0)
    lob = jnp.broadcast_to(lo_ref[...], (_EB, CHUNK)) == lax.broadcasted_iota(
        jnp.int32, (_EB, CHUNK), 1)
    part = jnp.dot(hia.astype(jnp.bfloat16), lob.astype(jnp.bfloat16),
                   preferred_element_type=jnp.float32)

    @pl.when(i == 0)
    def _():
        deg_ref[...] = jnp.zeros_like(deg_ref)

    deg_ref[...] += part


_BLK = 2048


def _dinv_of(dd_ref):
    return lax.rsqrt(dd_ref[...] + 1.0)


def _tca_body(x_ref, dd_ref, xh_ref):
    xh_ref[...] = x_ref[...] * _dinv_of(dd_ref)


def _tcb_body(p1_ref, xh_ref, dd_ref, w1_ref, b1_ref, gam_ref, bet_ref,
              ah_ref):
    dinv = _dinv_of(dd_ref)
    z1 = (p1_ref[0] + p1_ref[1] + xh_ref[...]) * dinv
    o1 = jnp.dot(z1, w1_ref[...], preferred_element_type=jnp.float32) + b1_ref[...]
    a = jnp.maximum(o1, 0.0) * gam_ref[...] + bet_ref[...]
    ah_ref[...] = a * dinv


def _tcc_body(p2_ref, ah_ref, dd_ref, w2_ref, b2_ref, o_ref):
    dinv = _dinv_of(dd_ref)
    z2 = (p2_ref[0] + p2_ref[1] + ah_ref[...]) * dinv
    o_ref[...] = jnp.dot(z2, w2_ref[...],
                         preferred_element_type=jnp.float32) + b2_ref[...]


def _row_spec(w):
    return pl.BlockSpec((_BLK, w), lambda i: (i, 0))


def _pair_spec(w):
    return pl.BlockSpec((2, _BLK, w), lambda i: (0, i, 0))


def _full_spec(r, ccols):
    return pl.BlockSpec((r, ccols), lambda i: (0, 0))


def kernel(x, edge_index, W1, b1, gamma, beta, W2, b2):
    src = edge_index[0]
    dst = edge_index[1]
    pad = EPAD - E
    src_p = jnp.concatenate([src, jnp.zeros((pad,), jnp.int32)]).reshape(NW, K, CHUNK)
    dst_p = jnp.concatenate([dst, jnp.full((pad,), N, jnp.int32)]).reshape(NW, K, CHUNK)
    hi_row = (dst >> 7).reshape(1, E)
    lo_col = (dst & 127).reshape(E, 1)

    z128 = jnp.zeros((NP // NS, D_HID), jnp.float32)

    x_pad = jnp.pad(x, ((0, NP - N), (0, 0)))
    gamma_eff = (gamma * np.float32(1.0 / np.sqrt(1.0 + BN_EPS))).reshape(1, D_HID)
    beta_r = beta.reshape(1, D_HID)
    b1_r = b1.reshape(1, D_HID)
    W2p = jnp.zeros((D_HID, W2P), jnp.float32).at[:, :D_OUT].set(W2)
    b2p = jnp.zeros((1, W2P), jnp.float32).at[0, :D_OUT].set(b2)

    deg2d = pl.pallas_call(
        _deg_body,
        grid=(E // _EB,),
        in_specs=[pl.BlockSpec((1, _EB), lambda i: (0, i)),
                  pl.BlockSpec((_EB, 1), lambda i: (i, 0))],
        out_specs=pl.BlockSpec((HI, CHUNK), lambda i: (0, 0)),
        out_shape=jax.ShapeDtypeStruct((HI, CHUNK), jnp.float32),
    )(hi_row, lo_col)
    dd = deg2d.reshape(NP, 1)

    grid = NP // _BLK
    col_spec = pl.BlockSpec((_BLK, 1), lambda i: (i, 0))

    xh = pl.pallas_call(
        _tca_body,
        grid=(grid,),
        in_specs=[_row_spec(D_IN), col_spec],
        out_specs=_row_spec(D_IN),
        out_shape=jax.ShapeDtypeStruct((NP, D_IN), jnp.float32),
    )(x_pad, dd)

    p1 = _make_scat(D_HID, NP)(xh, src_p, dst_p, z128)

    ah = pl.pallas_call(
        _tcb_body,
        grid=(grid,),
        in_specs=[_pair_spec(D_HID), _row_spec(D_HID), col_spec,
                  _full_spec(D_IN, D_HID), _full_spec(1, D_HID),
                  _full_spec(1, D_HID), _full_spec(1, D_HID)],
        out_specs=_row_spec(D_HID),
        out_shape=jax.ShapeDtypeStruct((NP, D_HID), jnp.float32),
    )(p1, xh, dd, W1, b1_r, gamma_eff, beta_r)

    p2 = _make_scat(D_HID, NP)(ah, src_p, dst_p, z128)

    o = pl.pallas_call(
        _tcc_body,
        grid=(grid,),
        in_specs=[_pair_spec(D_HID), _row_spec(D_HID), col_spec,
                  _full_spec(D_HID, W2P), _full_spec(1, W2P)],
        out_specs=_row_spec(W2P),
        out_shape=jax.ShapeDtypeStruct((NP, W2P), jnp.float32),
    )(p2, ah, dd, W2p, b2p)

    return o[:N, :D_OUT]
